# baseline (speedup 1.0000x reference)
import jax
import jax.numpy as jnp
from jax import lax
from jax.experimental import pallas as pl
from jax.experimental.pallas import tpu as pltpu
from jax.experimental.pallas import tpu_sc as plsc

_N = 5000
_PAD = 5120
_T = 0.5
_MAX_ROUNDS = 100
_NEG = -1e30
_NSUB = 16
_L = 16
_SLICE = _PAD // _NSUB
_CHUNKS = _SLICE // _L
_OROWS = _PAD // _NSUB


def _sc_body(x1h, y1h, x2h, y2h, sh, cnth, outh,
             x1v, y1v, x2v, y2v, sv_, cntv, alive, outv, stage, candv, perm,
             cands):
    sid = lax.axis_index("s")
    base = sid * _SLICE
    l16 = lax.iota(jnp.int32, 16)
    zeros16 = jnp.zeros((16,), jnp.float32)
    obase_v = jnp.full((16,), sid * _OROWS, jnp.int32)

    def bf_max(v):
        for k in (1, 2, 4, 8):
            perm[...] = v
            v = jnp.maximum(v, plsc.load_gather(perm, [l16 ^ k]))
        return v

    def bf_min(v):
        for k in (1, 2, 4, 8):
            perm[...] = v
            v = jnp.minimum(v, plsc.load_gather(perm, [l16 ^ k]))
        return v

    def bf_sum(v):
        for k in (1, 2, 4, 8):
            perm[...] = v
            v = v + plsc.load_gather(perm, [l16 ^ k])
        return v

    pltpu.sync_copy(x1h, x1v)
    pltpu.sync_copy(y1h, y1v)
    pltpu.sync_copy(x2h, x2v)
    pltpu.sync_copy(y2h, y2v)
    pltpu.sync_copy(sh, sv_)
    pltpu.sync_copy(cnth, cntv)
    cntvec = cntv[...]

    for j in range(_CHUNKS):
        iv = base + j * _L + l16
        alive[pl.ds(j * _L, _L)] = jnp.where(iv < _N, 1.0, 0.0)
    for r in range(_OROWS):
        outv[pl.ds(r * _L, _L)] = zeros16

    def round_body(t, carry):
        ipv, x1w, y1w, x2w, y2w, sw, aw = carry
        mv = jnp.full((16,), _NEG, jnp.float32)
        mivf = jnp.full((16,), 1e9, jnp.float32)
        prv = zeros16
        for j in range(_CHUNKS):
            sl = pl.ds(j * _L, _L)
            gsl = pl.ds(base + j * _L, _L)
            iv = base + j * _L + l16
            av = alive[sl]
            xv1 = x1v[gsl]
            yv1 = y1v[gsl]
            xv2 = x2v[gsl]
            yv2 = y2v[gsl]
            scv = sv_[gsl]
            w = jnp.maximum(jnp.minimum(xv2, x2w) - jnp.maximum(xv1, x1w), 0.0)
            h = jnp.maximum(jnp.minimum(yv2, y2w) - jnp.maximum(yv1, y1w), 0.0)
            inter = w * h
            areav = (xv2 - xv1) * (yv2 - yv1)
            denom = jnp.maximum(areav + aw - inter, 1e-9)
            anew = jnp.where(
                (iv != ipv) & ~(inter > _T * denom) & (av > 0.5), 1.0, 0.0)
            alive[sl] = anew
            hi = (scv > sw) | ((scv == sw) & (iv < ipv))
            prv = prv + jnp.where(hi, 1.0, 0.0)
            cand = jnp.where(anew > 0.5, scv, _NEG)
            upd = cand > mv
            mivf = jnp.where(upd, iv.astype(jnp.float32), mivf)
            mv = jnp.maximum(mv, cand)
        mloc = jnp.max(mv)
        ilocf = jnp.min(jnp.where(mv == mloc, mivf, jnp.float32(1e9)))
        prf = jnp.sum(prv)
        row = jnp.where(l16 == 0, mloc,
                        jnp.where(l16 == 1, ilocf,
                                  jnp.where(l16 == 2, prf, 0.0)))
        stage[...] = row
        pltpu.sync_copy(stage, cands.at[sid])
        plsc.subcore_barrier()
        pltpu.sync_copy(cands, candv)
        plsc.subcore_barrier()
        zi = jnp.zeros((16,), jnp.int32)
        sc16 = plsc.load_gather(candv, [l16, zi])
        ixf16 = plsc.load_gather(candv, [l16, zi + 1])
        pr16 = plsc.load_gather(candv, [l16, zi + 2])
        mg = jnp.max(sc16)
        iwf = jnp.min(jnp.where(sc16 == mg, ixf16, jnp.float32(1e9)))
        iw = jnp.full((16,), iwf.astype(jnp.int32))
        rank = jnp.full((16,), jnp.sum(pr16).astype(jnp.int32))
        tv = jnp.full((16,), t, jnp.int32)
        rec = (ipv < _N) & (tv - 1 < cntvec) & (rank >= obase_v) & \
              (rank < obase_v + _OROWS)
        rrow = jnp.where(l16 == 0, x1w,
                         jnp.where(l16 == 1, y1w,
                                   jnp.where(l16 == 2, x2w,
                                             jnp.where(l16 == 3, y2w,
                                                       jnp.where(l16 == 4, sw,
                                                                 0.0)))))
        ridx = jnp.where(rec, (rank - obase_v) * _L, 0) + l16
        plsc.store_scatter(outv, [ridx], rrow, mask=rec)
        iws = jnp.minimum(iw, _PAD - 1)
        x1n = plsc.load_gather(x1v, [iws])
        y1n = plsc.load_gather(y1v, [iws])
        x2n = plsc.load_gather(x2v, [iws])
        y2n = plsc.load_gather(y2v, [iws])
        sn = plsc.load_gather(sv_, [iws])
        an = (x2n - x1n) * (y2n - y1n)
        return iw, x1n, y1n, x2n, y2n, sn, an

    big = jnp.full((16,), 3e38, jnp.float32)
    bigi = jnp.full((16,), 2**30, jnp.int32)
    carry = (bigi, zeros16, zeros16, zeros16, zeros16, big, zeros16)
    lax.fori_loop(0, _MAX_ROUNDS + 1, round_body, carry)

    pltpu.sync_copy(outv, outh.at[pl.ds(sid * _OROWS * _L, _OROWS * _L)])


@jax.jit
def _nms_sc(boxes, scores, counts):
    pad = _PAD - _N
    x1 = jnp.pad(boxes[:, 0], (0, pad))
    y1 = jnp.pad(boxes[:, 1], (0, pad))
    x2 = jnp.pad(boxes[:, 2], (0, pad))
    y2 = jnp.pad(boxes[:, 3], (0, pad))
    s = jnp.pad(scores, (0, pad), constant_values=_NEG)
    cnt = jnp.full((16,), jnp.asarray(counts, jnp.int32))
    mesh = plsc.VectorSubcoreMesh(core_axis_name="c", subcore_axis_name="s",
                                  num_cores=1)
    f = pl.kernel(
        _sc_body,
        out_type=jax.ShapeDtypeStruct((_PAD * _L,), jnp.float32),
        mesh=mesh,
        compiler_params=pltpu.CompilerParams(needs_layout_passes=False,
                                             use_tc_tiling_on_sc=False),
        scratch_types=[
            pltpu.VMEM((_PAD,), jnp.float32),
            pltpu.VMEM((_PAD,), jnp.float32),
            pltpu.VMEM((_PAD,), jnp.float32),
            pltpu.VMEM((_PAD,), jnp.float32),
            pltpu.VMEM((_PAD,), jnp.float32),
            pltpu.VMEM((16,), jnp.int32),
            pltpu.VMEM((_SLICE,), jnp.float32),
            pltpu.VMEM((_OROWS * _L,), jnp.float32),
            pltpu.VMEM((16,), jnp.float32),
            pltpu.VMEM((16, 16), jnp.float32),
            pltpu.VMEM((16,), jnp.float32),
            pltpu.VMEM_SHARED((16, 16), jnp.float32),
        ],
    )
    out = f(x1, y1, x2, y2, s, cnt)
    return out.reshape(_PAD, _L)[:_N, :5]


def kernel(boxes, scores, counts):
    return _nms_sc(boxes, scores, counts)


# speedup vs baseline: 1.3009x; 1.3009x over previous
import jax
import jax.numpy as jnp
from jax import lax
from jax.experimental import pallas as pl
from jax.experimental.pallas import tpu as pltpu
from jax.experimental.pallas import tpu_sc as plsc

_N = 5000
_PAD = 5120
_T = 0.5
_MAX_ROUNDS = 100
_NEG = -1e30
_NSUB = 16
_L = 16
_SLICE = _PAD // _NSUB
_CHUNKS = _SLICE // _L
_OROWS = _PAD // _NSUB


def _sc_body(x1h, y1h, x2h, y2h, sh, cnth, outh,
             x1v, y1v, x2v, y2v, sv_, cntv, alive, outv, stage, candv,
             cands):
    sid = lax.axis_index("s")
    base = sid * _SLICE
    l16 = lax.iota(jnp.int32, 16)
    zeros16 = jnp.zeros((16,), jnp.float32)
    obase_v = jnp.full((16,), sid * _OROWS, jnp.int32)

    gs = pl.ds(base, _SLICE)
    pltpu.sync_copy(x1h.at[gs], x1v)
    pltpu.sync_copy(y1h.at[gs], y1v)
    pltpu.sync_copy(x2h.at[gs], x2v)
    pltpu.sync_copy(y2h.at[gs], y2v)
    pltpu.sync_copy(sh.at[gs], sv_)
    pltpu.sync_copy(cnth, cntv)
    cntvec = cntv[...]

    for j in range(_CHUNKS):
        iv = base + j * _L + l16
        alive[pl.ds(j * _L, _L)] = jnp.where(iv < _N, 1.0, 0.0)
    for r in range(_OROWS):
        outv[pl.ds(r * _L, _L)] = zeros16

    def round_body(t, carry):
        ipv, x1w, y1w, x2w, y2w, sw, aw = carry
        vwv = ipv < _N
        mv = jnp.full((16,), _NEG, jnp.float32)
        mivf = jnp.full((16,), 1e9, jnp.float32)
        prv = zeros16
        for j in range(_CHUNKS):
            sl = pl.ds(j * _L, _L)
            iv = base + j * _L + l16
            av = alive[sl]
            xv1 = x1v[sl]
            yv1 = y1v[sl]
            xv2 = x2v[sl]
            yv2 = y2v[sl]
            scv = sv_[sl]
            w = jnp.maximum(jnp.minimum(xv2, x2w) - jnp.maximum(xv1, x1w), 0.0)
            h = jnp.maximum(jnp.minimum(yv2, y2w) - jnp.maximum(yv1, y1w), 0.0)
            inter = w * h
            areav = (xv2 - xv1) * (yv2 - yv1)
            denom = jnp.maximum(areav + aw - inter, 1e-9)
            anew = jnp.where(
                (iv != ipv) & ~((inter > _T * denom) & vwv) & (av > 0.5),
                1.0, 0.0)
            alive[sl] = anew
            hi = (scv > sw) | ((scv == sw) & (iv < ipv))
            prv = prv + jnp.where(hi, 1.0, 0.0)
            cand = jnp.where(anew > 0.5, scv, _NEG)
            upd = cand > mv
            mivf = jnp.where(upd, iv.astype(jnp.float32), mivf)
            mv = jnp.maximum(mv, cand)
        mloc = jnp.max(mv)
        ilocf = jnp.min(jnp.where(mv == mloc, mivf, jnp.float32(1e9)))
        prf = jnp.sum(prv)
        li = jnp.clip(jnp.full((16,), ilocf.astype(jnp.int32)) - base,
                      0, _SLICE - 1)
        xc1 = plsc.load_gather(x1v, [li])
        yc1 = plsc.load_gather(y1v, [li])
        xc2 = plsc.load_gather(x2v, [li])
        yc2 = plsc.load_gather(y2v, [li])
        row = jnp.where(l16 == 0, mloc,
                        jnp.where(l16 == 1, ilocf,
                                  jnp.where(l16 == 2, prf,
                                            jnp.where(l16 == 3, xc1,
                                                      jnp.where(l16 == 4, yc1,
                                                                jnp.where(l16 == 5, xc2,
                                                                          jnp.where(l16 == 6, yc2,
                                                                                    0.0)))))))
        stage[...] = row
        p = t & 1
        pltpu.sync_copy(stage, cands.at[p, sid])
        plsc.subcore_barrier()
        pltpu.sync_copy(cands.at[p], candv)
        zi = jnp.zeros((16,), jnp.int32)
        sc16 = plsc.load_gather(candv, [l16, zi])
        ixf16 = plsc.load_gather(candv, [l16, zi + 1])
        pr16 = plsc.load_gather(candv, [l16, zi + 2])
        mg = jnp.max(sc16)
        winm = sc16 == mg
        iwf = jnp.min(jnp.where(winm, ixf16, jnp.float32(1e9)))
        iw = jnp.full((16,), iwf.astype(jnp.int32))
        rank = jnp.full((16,), jnp.sum(pr16).astype(jnp.int32))
        l16f = l16.astype(jnp.float32)
        wlf = jnp.min(jnp.where(winm & (ixf16 == iwf), l16f,
                                jnp.float32(1e9)))
        wlv = jnp.full((16,), wlf.astype(jnp.int32))
        tv = jnp.full((16,), t, jnp.int32)
        rec = vwv & (tv - 1 < cntvec) & (rank >= obase_v) & \
              (rank < obase_v + _OROWS)
        rrow = jnp.where(l16 == 0, x1w,
                         jnp.where(l16 == 1, y1w,
                                   jnp.where(l16 == 2, x2w,
                                             jnp.where(l16 == 3, y2w,
                                                       jnp.where(l16 == 4, sw,
                                                                 0.0)))))
        ridx = jnp.where(rec, (rank - obase_v) * _L, 0) + l16
        plsc.store_scatter(outv, [ridx], rrow, mask=rec)
        x1n = plsc.load_gather(candv, [wlv, zi + 3])
        y1n = plsc.load_gather(candv, [wlv, zi + 4])
        x2n = plsc.load_gather(candv, [wlv, zi + 5])
        y2n = plsc.load_gather(candv, [wlv, zi + 6])
        sn = jnp.full((16,), mg)
        an = (x2n - x1n) * (y2n - y1n)
        return iw, x1n, y1n, x2n, y2n, sn, an

    big = jnp.full((16,), 3e38, jnp.float32)
    bigi = jnp.full((16,), 2**30, jnp.int32)
    carry = (bigi, zeros16, zeros16, zeros16, zeros16, big, zeros16)
    lax.fori_loop(0, _MAX_ROUNDS + 1, round_body, carry)

    pltpu.sync_copy(outv, outh.at[pl.ds(sid * _OROWS * _L, _OROWS * _L)])


@jax.jit
def _nms_sc(boxes, scores, counts):
    pad = _PAD - _N
    x1 = jnp.pad(boxes[:, 0], (0, pad))
    y1 = jnp.pad(boxes[:, 1], (0, pad))
    x2 = jnp.pad(boxes[:, 2], (0, pad))
    y2 = jnp.pad(boxes[:, 3], (0, pad))
    s = jnp.pad(scores, (0, pad), constant_values=_NEG)
    cnt = jnp.full((16,), jnp.asarray(counts, jnp.int32))
    mesh = plsc.VectorSubcoreMesh(core_axis_name="c", subcore_axis_name="s",
                                  num_cores=1)
    f = pl.kernel(
        _sc_body,
        out_type=jax.ShapeDtypeStruct((_PAD * _L,), jnp.float32),
        mesh=mesh,
        compiler_params=pltpu.CompilerParams(needs_layout_passes=False,
                                             use_tc_tiling_on_sc=False),
        scratch_types=[
            pltpu.VMEM((_SLICE,), jnp.float32),
            pltpu.VMEM((_SLICE,), jnp.float32),
            pltpu.VMEM((_SLICE,), jnp.float32),
            pltpu.VMEM((_SLICE,), jnp.float32),
            pltpu.VMEM((_SLICE,), jnp.float32),
            pltpu.VMEM((16,), jnp.int32),
            pltpu.VMEM((_SLICE,), jnp.float32),
            pltpu.VMEM((_OROWS * _L,), jnp.float32),
            pltpu.VMEM((16,), jnp.float32),
            pltpu.VMEM((16, 16), jnp.float32),
            pltpu.VMEM_SHARED((2, 16, 16), jnp.float32),
        ],
    )
    out = f(x1, y1, x2, y2, s, cnt)
    return out.reshape(_PAD, _L)[:_N, :5]


def kernel(boxes, scores, counts):
    return _nms_sc(boxes, scores, counts)
